# Initial kernel scaffold; baseline (speedup 1.0000x reference)
import functools

import jax
import jax.numpy as jnp
from jax import lax
from jax.experimental import pallas as pl
from jax.experimental.pallas import tpu as pltpu
from jax.experimental.pallas import tpu_sc as plsc

_USER = 5000
_ITEM = 5000
_N = _USER + _ITEM
_D = 128
_E = 320000

_NC = 2
_NS = 16
_NW = _NC * _NS
_EPW = _E // _NW
_CH = 80
_NCHUNK = _EPW // _CH
_RPS = _N // _NS
_ZR = 125

_mesh = plsc.VectorSubcoreMesh(core_axis_name="c", subcore_axis_name="s")


@functools.partial(
    pl.kernel,
    out_type=jax.ShapeDtypeStruct((_NC, _N, _D), jnp.float32),
    mesh=_mesh,
    scratch_types=[
        pltpu.VMEM((_NCHUNK, _CH), jnp.int32),
        pltpu.VMEM((_NCHUNK, _CH), jnp.int32),
        pltpu.VMEM((_NCHUNK, _CH), jnp.float32),
        pltpu.VMEM((_CH, _D), jnp.float32),
        pltpu.VMEM((_ZR, _D), jnp.float32),
        pltpu.VMEM_SHARED((_N, _D), jnp.float32),
        pltpu.SemaphoreType.DMA,
    ],
)
def _spmm(table, colsr, rowsr, wr, out, cols_v, rows_v, w_v, gbuf, zbuf,
          acc, sem):
    c = lax.axis_index("c")
    s = lax.axis_index("s")
    wid = c * _NS + s

    zero16 = jnp.zeros((16,), jnp.float32)

    def zrow(i, _):
        for j in range(_D // 16):
            zbuf[i, pl.ds(j * 16, 16)] = zero16
        return 0

    lax.fori_loop(0, _ZR, zrow, 0)
    for k in range(_RPS // _ZR):
        pltpu.sync_copy(zbuf, acc.at[pl.ds(s * _RPS + k * _ZR, _ZR)])
    plsc.subcore_barrier()

    pltpu.sync_copy(colsr.at[wid], cols_v)
    pltpu.sync_copy(rowsr.at[wid], rows_v)
    pltpu.sync_copy(wr.at[wid], w_v)

    def chunk(ch, _):
        pltpu.async_copy(table.at[cols_v.at[ch]], gbuf, sem).wait()

        def edge(e, _):
            wsc = w_v[ch, e]
            for j in range(_D // 16):
                sl = pl.ds(j * 16, 16)
                gbuf[e, sl] = gbuf[e, sl] * wsc
            return 0

        lax.fori_loop(0, _CH, edge, 0)

        pltpu.sync_copy(gbuf, acc.at[rows_v.at[ch]], add=True)
        return 0

    lax.fori_loop(0, _NCHUNK, chunk, 0)

    plsc.subcore_barrier()
    pltpu.sync_copy(acc.at[pl.ds(s * _RPS, _RPS)],
                    out.at[c, pl.ds(s * _RPS, _RPS)])


def _combine_body(p_ref, o_ref):
    o_ref[...] = p_ref[0] + p_ref[1]


_combine = pl.pallas_call(
    _combine_body,
    grid=(10,),
    in_specs=[pl.BlockSpec((2, _N // 10, _D), lambda i: (0, i, 0))],
    out_specs=pl.BlockSpec((_N // 10, _D), lambda i: (i, 0)),
    out_shape=jax.ShapeDtypeStruct((_N, _D), jnp.float32),
)


def _final_body(e0, l1, p2, uh, ih, out_e, out_h):
    ssum = e0[...] + l1[...] + p2[0] + p2[1]
    out_e[...] = ssum
    i = pl.program_id(0)
    gu = jnp.dot(uh[...].T, uh[...], precision=lax.Precision.HIGHEST,
                 preferred_element_type=jnp.float32)
    gi = jnp.dot(ih[...].T, ih[...], precision=lax.Precision.HIGHEST,
                 preferred_element_type=jnp.float32)
    g = jnp.where(i == 0, gu, gi)
    out_h[...] = jnp.dot(ssum, g, precision=lax.Precision.HIGHEST,
                         preferred_element_type=jnp.float32)


_final = pl.pallas_call(
    _final_body,
    grid=(2,),
    in_specs=[
        pl.BlockSpec((_USER, _D), lambda i: (i, 0)),
        pl.BlockSpec((_USER, _D), lambda i: (i, 0)),
        pl.BlockSpec((2, _USER, _D), lambda i: (0, i, 0)),
        pl.BlockSpec((_D, _D), lambda i: (0, 0)),
        pl.BlockSpec((_D, _D), lambda i: (0, 0)),
    ],
    out_specs=[
        pl.BlockSpec((_USER, _D), lambda i: (i, 0)),
        pl.BlockSpec((_USER, _D), lambda i: (i, 0)),
    ],
    out_shape=[
        jax.ShapeDtypeStruct((_N, _D), jnp.float32),
        jax.ShapeDtypeStruct((_N, _D), jnp.float32),
    ],
)


@jax.jit
def kernel(edge_index, edge_weight, uEmbeds, iEmbeds, uHyper, iHyper):
    table0 = jnp.concatenate([uEmbeds, iEmbeds], axis=0)
    rows = edge_index[0].astype(jnp.int32).reshape(_NW, _NCHUNK, _CH)
    cols = edge_index[1].astype(jnp.int32).reshape(_NW, _NCHUNK, _CH)
    wr = edge_weight.reshape(_NW, _NCHUNK, _CH)

    p1 = _spmm(table0, cols, rows, wr)
    lat1 = _combine(p1)
    p2 = _spmm(lat1, cols, rows, wr)
    out_e, out_h = _final(table0, lat1, p2, uHyper, iHyper)
    return out_e, out_h[:_USER], out_h[_USER:]


# speedup vs baseline: 5.9136x; 5.9136x over previous
import functools

import jax
import jax.numpy as jnp
from jax import lax
from jax.experimental import pallas as pl
from jax.experimental.pallas import tpu as pltpu
from jax.experimental.pallas import tpu_sc as plsc

_USER = 5000
_ITEM = 5000
_N = _USER + _ITEM
_D = 128
_E = 320000

_NC = 2
_NS = 16
_NW = _NC * _NS
_EPW = _E // _NW
_CH = 80
_NSTG = 5
_CPS = 25
_NCHUNK = _NSTG * _CPS
_RPS = 624
_TAIL = _N - _NS * _RPS
_ZC = 48

_mesh = plsc.VectorSubcoreMesh(core_axis_name="c", subcore_axis_name="s")


@functools.partial(
    pl.kernel,
    out_type=jax.ShapeDtypeStruct((_NC, _N, _D), jnp.float32),
    mesh=_mesh,
    scratch_types=[
        pltpu.VMEM((_CPS, _CH), jnp.int32),
        pltpu.VMEM((_CPS, _CH), jnp.int32),
        pltpu.VMEM((_CPS, _CH), jnp.float32),
        pltpu.VMEM((_CH, _D), jnp.float32),
        pltpu.VMEM_SHARED((_N, _D), jnp.float32),
        pltpu.SemaphoreType.DMA,
    ],
)
def _spmm(table, colsr, rowsr, wr, out, cols_v, rows_v, w_v, gbuf,
          acc, sem):
    c = lax.axis_index("c")
    s = lax.axis_index("s")
    wid = c * _NS + s

    zero16 = jnp.zeros((16,), jnp.float32)

    def zrow(i, _):
        for j in range(_D // 16):
            gbuf[i, pl.ds(j * 16, 16)] = zero16
        return 0

    lax.fori_loop(0, _CH, zrow, 0)
    for k in range(_RPS // _ZC):
        pltpu.sync_copy(gbuf.at[pl.ds(0, _ZC)],
                        acc.at[pl.ds(s * _RPS + k * _ZC, _ZC)])

    @pl.when(s == 0)
    def _():
        pltpu.sync_copy(gbuf.at[pl.ds(0, _TAIL)],
                        acc.at[pl.ds(_NS * _RPS, _TAIL)])

    plsc.subcore_barrier()

    def stage(st, _):
        pltpu.sync_copy(colsr.at[wid, st], cols_v)
        pltpu.sync_copy(rowsr.at[wid, st], rows_v)
        pltpu.sync_copy(wr.at[wid, st], w_v)

        def chunk(ch, _):
            pltpu.async_copy(table.at[cols_v.at[ch]], gbuf, sem).wait()

            def egroup(g, _):
                wv = w_v[ch, pl.ds(g * 16, 16)]
                for l in range(16):
                    wsc = wv[l]
                    e = g * 16 + l
                    for j in range(_D // 16):
                        sl = pl.ds(j * 16, 16)
                        gbuf[e, sl] = gbuf[e, sl] * wsc
                return 0

            lax.fori_loop(0, _CH // 16, egroup, 0)

            pltpu.sync_copy(gbuf, acc.at[rows_v.at[ch]], add=True)
            return 0

        lax.fori_loop(0, _CPS, chunk, 0)
        return 0

    lax.fori_loop(0, _NSTG, stage, 0)

    plsc.subcore_barrier()
    pltpu.sync_copy(acc.at[pl.ds(s * _RPS, _RPS)],
                    out.at[c, pl.ds(s * _RPS, _RPS)])

    @pl.when(s == 0)
    def _():
        pltpu.sync_copy(acc.at[pl.ds(_NS * _RPS, _TAIL)],
                        out.at[c, pl.ds(_NS * _RPS, _TAIL)])


def _combine_body(p_ref, o_ref):
    o_ref[...] = p_ref[0] + p_ref[1]


_combine = pl.pallas_call(
    _combine_body,
    grid=(10,),
    in_specs=[pl.BlockSpec((2, _N // 10, _D), lambda i: (0, i, 0))],
    out_specs=pl.BlockSpec((_N // 10, _D), lambda i: (i, 0)),
    out_shape=jax.ShapeDtypeStruct((_N, _D), jnp.float32),
)


def _final_body(e0, l1, p2, uh, ih, out_e, out_h):
    ssum = e0[...] + l1[...] + p2[0] + p2[1]
    out_e[...] = ssum
    i = pl.program_id(0)
    gu = jnp.dot(uh[...].T, uh[...], precision=lax.Precision.HIGHEST,
                 preferred_element_type=jnp.float32)
    gi = jnp.dot(ih[...].T, ih[...], precision=lax.Precision.HIGHEST,
                 preferred_element_type=jnp.float32)
    g = jnp.where(i == 0, gu, gi)
    out_h[...] = jnp.dot(ssum, g, precision=lax.Precision.HIGHEST,
                         preferred_element_type=jnp.float32)


_final = pl.pallas_call(
    _final_body,
    grid=(2,),
    in_specs=[
        pl.BlockSpec((_USER, _D), lambda i: (i, 0)),
        pl.BlockSpec((_USER, _D), lambda i: (i, 0)),
        pl.BlockSpec((2, _USER, _D), lambda i: (0, i, 0)),
        pl.BlockSpec((_D, _D), lambda i: (0, 0)),
        pl.BlockSpec((_D, _D), lambda i: (0, 0)),
    ],
    out_specs=[
        pl.BlockSpec((_USER, _D), lambda i: (i, 0)),
        pl.BlockSpec((_USER, _D), lambda i: (i, 0)),
    ],
    out_shape=[
        jax.ShapeDtypeStruct((_N, _D), jnp.float32),
        jax.ShapeDtypeStruct((_N, _D), jnp.float32),
    ],
)


@jax.jit
def kernel(edge_index, edge_weight, uEmbeds, iEmbeds, uHyper, iHyper):
    table0 = jnp.concatenate([uEmbeds, iEmbeds], axis=0)
    rows = edge_index[0].astype(jnp.int32).reshape(_NW, _NSTG, _CPS, _CH)
    cols = edge_index[1].astype(jnp.int32).reshape(_NW, _NSTG, _CPS, _CH)
    wr = edge_weight.reshape(_NW, _NSTG, _CPS, _CH)

    p1 = _spmm(table0, cols, rows, wr)
    lat1 = _combine(p1)
    p2 = _spmm(lat1, cols, rows, wr)
    out_e, out_h = _final(table0, lat1, p2, uHyper, iHyper)
    return out_e, out_h[:_USER], out_h[_USER:]


# speedup vs baseline: 9.1974x; 1.5553x over previous
import functools

import jax
import jax.numpy as jnp
from jax import lax
from jax.experimental import pallas as pl
from jax.experimental.pallas import tpu as pltpu
from jax.experimental.pallas import tpu_sc as plsc

_USER = 5000
_ITEM = 5000
_N = _USER + _ITEM
_D = 128
_E = 320000

_NC = 2
_NS = 16
_NW = _NC * _NS
_EPW = _E // _NW
_CH = 80
_NSTG = 5
_CPS = 25
_NCHUNK = _NSTG * _CPS
_RPS = 624
_TAIL = _N - _NS * _RPS
_ZC = 48

_mesh = plsc.VectorSubcoreMesh(core_axis_name="c", subcore_axis_name="s")


@functools.partial(
    pl.kernel,
    out_type=jax.ShapeDtypeStruct((_NC, _N, _D), jnp.float32),
    mesh=_mesh,
    scratch_types=[
        pltpu.VMEM((_CPS, _CH), jnp.int32),
        pltpu.VMEM((_CPS, _CH), jnp.int32),
        pltpu.VMEM((_CPS, _CH), jnp.float32),
        pltpu.VMEM((_CH, _D), jnp.float32),
        pltpu.VMEM((_CH, _D), jnp.float32),
        pltpu.VMEM_SHARED((_N, _D), jnp.float32),
        pltpu.SemaphoreType.DMA,
        pltpu.SemaphoreType.DMA,
        pltpu.SemaphoreType.DMA,
        pltpu.SemaphoreType.DMA,
    ],
)
def _spmm(table, colsr, rowsr, wr, out, cols_v, rows_v, w_v, gb0, gb1,
          acc, gsem0, gsem1, ssem0, ssem1):
    c = lax.axis_index("c")
    s = lax.axis_index("s")
    wid = c * _NS + s
    gbufs = (gb0, gb1)
    gsems = (gsem0, gsem1)
    ssems = (ssem0, ssem1)

    zero16 = jnp.zeros((16,), jnp.float32)

    def zrow(i, _):
        for j in range(_D // 16):
            gb0[i, pl.ds(j * 16, 16)] = zero16
        return 0

    lax.fori_loop(0, _CH, zrow, 0)
    for k in range(_RPS // _ZC):
        pltpu.sync_copy(gb0.at[pl.ds(0, _ZC)],
                        acc.at[pl.ds(s * _RPS + k * _ZC, _ZC)])

    @pl.when(s == 0)
    def _():
        pltpu.sync_copy(gb0.at[pl.ds(0, _TAIL)],
                        acc.at[pl.ds(_NS * _RPS, _TAIL)])

    plsc.subcore_barrier()

    def scale(ch, gb):
        def egroup(g, _):
            wv = w_v[ch, pl.ds(g * 16, 16)]
            for l in range(16):
                wsc = wv[l]
                e = g * 16 + l
                for j in range(_D // 16):
                    sl = pl.ds(j * 16, 16)
                    gb[e, sl] = gb[e, sl] * wsc
            return 0

        lax.fori_loop(0, _CH // 16, egroup, 0)

    for st in range(_NSTG):
        pltpu.sync_copy(colsr.at[wid, st], cols_v)
        pltpu.sync_copy(rowsr.at[wid, st], rows_v)
        pltpu.sync_copy(wr.at[wid, st], w_v)

        pltpu.async_copy(table.at[cols_v.at[0]], gb0, gsem0)

        def chunk(ch, _):
            nxt = ch + 1
            for b in range(2):
                @pl.when(jnp.logical_and(nxt < _CPS, (nxt & 1) == b))
                def _():
                    @pl.when(nxt >= 2)
                    def _():
                        pltpu.make_async_copy(
                            gbufs[b], acc.at[rows_v.at[0]],
                            ssems[b]).wait()
                    pltpu.async_copy(table.at[cols_v.at[nxt]], gbufs[b],
                                     gsems[b])
            for b in range(2):
                @pl.when((ch & 1) == b)
                def _():
                    pltpu.make_async_copy(table.at[cols_v.at[ch]],
                                          gbufs[b], gsems[b]).wait()
                    scale(ch, gbufs[b])
                    pltpu.async_copy(gbufs[b], acc.at[rows_v.at[ch]],
                                     ssems[b], add=True)
            return 0

        lax.fori_loop(0, _CPS, chunk, 0)
        pltpu.make_async_copy(gb0, acc.at[rows_v.at[0]], ssem0).wait()
        pltpu.make_async_copy(gb1, acc.at[rows_v.at[0]], ssem1).wait()

    plsc.subcore_barrier()
    pltpu.sync_copy(acc.at[pl.ds(s * _RPS, _RPS)],
                    out.at[c, pl.ds(s * _RPS, _RPS)])

    @pl.when(s == 0)
    def _():
        pltpu.sync_copy(acc.at[pl.ds(_NS * _RPS, _TAIL)],
                        out.at[c, pl.ds(_NS * _RPS, _TAIL)])


def _combine_body(p_ref, o_ref):
    o_ref[...] = p_ref[0] + p_ref[1]


_combine = pl.pallas_call(
    _combine_body,
    grid=(10,),
    in_specs=[pl.BlockSpec((2, _N // 10, _D), lambda i: (0, i, 0))],
    out_specs=pl.BlockSpec((_N // 10, _D), lambda i: (i, 0)),
    out_shape=jax.ShapeDtypeStruct((_N, _D), jnp.float32),
)


def _final_body(e0, l1, p2, uh, ih, out_e, out_h):
    ssum = e0[...] + l1[...] + p2[0] + p2[1]
    out_e[...] = ssum
    i = pl.program_id(0)
    gu = jnp.dot(uh[...].T, uh[...], precision=lax.Precision.HIGHEST,
                 preferred_element_type=jnp.float32)
    gi = jnp.dot(ih[...].T, ih[...], precision=lax.Precision.HIGHEST,
                 preferred_element_type=jnp.float32)
    g = jnp.where(i == 0, gu, gi)
    out_h[...] = jnp.dot(ssum, g, precision=lax.Precision.HIGHEST,
                         preferred_element_type=jnp.float32)


_final = pl.pallas_call(
    _final_body,
    grid=(2,),
    in_specs=[
        pl.BlockSpec((_USER, _D), lambda i: (i, 0)),
        pl.BlockSpec((_USER, _D), lambda i: (i, 0)),
        pl.BlockSpec((2, _USER, _D), lambda i: (0, i, 0)),
        pl.BlockSpec((_D, _D), lambda i: (0, 0)),
        pl.BlockSpec((_D, _D), lambda i: (0, 0)),
    ],
    out_specs=[
        pl.BlockSpec((_USER, _D), lambda i: (i, 0)),
        pl.BlockSpec((_USER, _D), lambda i: (i, 0)),
    ],
    out_shape=[
        jax.ShapeDtypeStruct((_N, _D), jnp.float32),
        jax.ShapeDtypeStruct((_N, _D), jnp.float32),
    ],
)


@jax.jit
def kernel(edge_index, edge_weight, uEmbeds, iEmbeds, uHyper, iHyper):
    table0 = jnp.concatenate([uEmbeds, iEmbeds], axis=0)
    rows = edge_index[0].astype(jnp.int32).reshape(_NW, _NSTG, _CPS, _CH)
    cols = edge_index[1].astype(jnp.int32).reshape(_NW, _NSTG, _CPS, _CH)
    wr = edge_weight.reshape(_NW, _NSTG, _CPS, _CH)

    p1 = _spmm(table0, cols, rows, wr)
    lat1 = _combine(p1)
    p2 = _spmm(lat1, cols, rows, wr)
    out_e, out_h = _final(table0, lat1, p2, uHyper, iHyper)
    return out_e, out_h[:_USER], out_h[_USER:]


# speedup vs baseline: 10.1056x; 1.0988x over previous
import functools

import jax
import jax.numpy as jnp
from jax import lax
from jax.experimental import pallas as pl
from jax.experimental.pallas import tpu as pltpu
from jax.experimental.pallas import tpu_sc as plsc

_USER = 5000
_ITEM = 5000
_N = _USER + _ITEM
_D = 128
_E = 320000

_NC = 2
_NS = 16
_NW = _NC * _NS
_EPW = _E // _NW
_CH = 80
_NSTG = 5
_CPS = 25
_NCHUNK = _NSTG * _CPS
_RPS = 624
_TAIL = _N - _NS * _RPS
_ZC = 48

_mesh = plsc.VectorSubcoreMesh(core_axis_name="c", subcore_axis_name="s")


@functools.partial(
    pl.kernel,
    out_type=jax.ShapeDtypeStruct((_NC, _N, _D), jnp.float32),
    mesh=_mesh,
    scratch_types=[
        pltpu.VMEM((_CPS, _CH), jnp.int32),
        pltpu.VMEM((_CPS, _CH), jnp.int32),
        pltpu.VMEM((_CPS, _CH), jnp.float32),
        pltpu.VMEM((_CH, _D), jnp.float32),
        pltpu.VMEM((_CH, _D), jnp.float32),
        pltpu.VMEM((_CH, _D), jnp.float32),
        pltpu.VMEM_SHARED((_N, _D), jnp.float32),
        pltpu.SemaphoreType.DMA,
        pltpu.SemaphoreType.DMA,
        pltpu.SemaphoreType.DMA,
        pltpu.SemaphoreType.DMA,
        pltpu.SemaphoreType.DMA,
        pltpu.SemaphoreType.DMA,
    ],
)
def _spmm(table, colsr, rowsr, wr, out, cols_v, rows_v, w_v, gb0, gb1, gb2,
          acc, gsem0, gsem1, gsem2, ssem0, ssem1, ssem2):
    c = lax.axis_index("c")
    s = lax.axis_index("s")
    wid = c * _NS + s
    gbufs = (gb0, gb1, gb2)
    gsems = (gsem0, gsem1, gsem2)
    ssems = (ssem0, ssem1, ssem2)

    zero16 = jnp.zeros((16,), jnp.float32)

    def zrow(i, _):
        for j in range(_D // 16):
            gb0[i, pl.ds(j * 16, 16)] = zero16
        return 0

    lax.fori_loop(0, _CH, zrow, 0)
    for k in range(_RPS // _ZC):
        pltpu.sync_copy(gb0.at[pl.ds(0, _ZC)],
                        acc.at[pl.ds(s * _RPS + k * _ZC, _ZC)])

    @pl.when(s == 0)
    def _():
        pltpu.sync_copy(gb0.at[pl.ds(0, _TAIL)],
                        acc.at[pl.ds(_NS * _RPS, _TAIL)])

    plsc.subcore_barrier()

    def scale(ch, gb):
        def egroup(g, _):
            wv = w_v[ch, pl.ds(g * 16, 16)]
            for l in range(16):
                wsc = wv[l]
                e = g * 16 + l
                for j in range(_D // 16):
                    sl = pl.ds(j * 16, 16)
                    gb[e, sl] = gb[e, sl] * wsc
            return 0

        lax.fori_loop(0, _CH // 16, egroup, 0)

    for st in range(_NSTG):
        pltpu.sync_copy(colsr.at[wid, st], cols_v)
        pltpu.sync_copy(rowsr.at[wid, st], rows_v)
        pltpu.sync_copy(wr.at[wid, st], w_v)

        pltpu.async_copy(table.at[cols_v.at[0]], gb0, gsem0)
        pltpu.async_copy(table.at[cols_v.at[1]], gb1, gsem1)

        def chunk(ch, _):
            cur = lax.rem(ch, 3)
            for b in range(3):
                @pl.when(cur == b)
                def _():
                    pltpu.make_async_copy(table.at[cols_v.at[ch]],
                                          gbufs[b], gsems[b]).wait()
                    scale(ch, gbufs[b])
                    pltpu.async_copy(gbufs[b], acc.at[rows_v.at[ch]],
                                     ssems[b], add=True)
            nxt = ch + 2
            nr = lax.rem(nxt, 3)
            for b in range(3):
                @pl.when(jnp.logical_and(nxt < _CPS, nr == b))
                def _():
                    @pl.when(nxt >= 3)
                    def _():
                        pltpu.make_async_copy(
                            gbufs[b], acc.at[rows_v.at[0]],
                            ssems[b]).wait()
                    pltpu.async_copy(table.at[cols_v.at[nxt]], gbufs[b],
                                     gsems[b])
            return 0

        lax.fori_loop(0, _CPS, chunk, 0)
        pltpu.make_async_copy(gb0, acc.at[rows_v.at[0]], ssem0).wait()
        pltpu.make_async_copy(gb1, acc.at[rows_v.at[0]], ssem1).wait()
        pltpu.make_async_copy(gb2, acc.at[rows_v.at[0]], ssem2).wait()

    plsc.subcore_barrier()
    pltpu.sync_copy(acc.at[pl.ds(s * _RPS, _RPS)],
                    out.at[c, pl.ds(s * _RPS, _RPS)])

    @pl.when(s == 0)
    def _():
        pltpu.sync_copy(acc.at[pl.ds(_NS * _RPS, _TAIL)],
                        out.at[c, pl.ds(_NS * _RPS, _TAIL)])


def _combine_body(p_ref, o_ref):
    o_ref[...] = p_ref[0] + p_ref[1]


_combine = pl.pallas_call(
    _combine_body,
    grid=(10,),
    in_specs=[pl.BlockSpec((2, _N // 10, _D), lambda i: (0, i, 0))],
    out_specs=pl.BlockSpec((_N // 10, _D), lambda i: (i, 0)),
    out_shape=jax.ShapeDtypeStruct((_N, _D), jnp.float32),
)


def _final_body(e0, l1, p2, uh, ih, out_e, out_h):
    ssum = e0[...] + l1[...] + p2[0] + p2[1]
    out_e[...] = ssum
    i = pl.program_id(0)
    gu = jnp.dot(uh[...].T, uh[...], precision=lax.Precision.HIGHEST,
                 preferred_element_type=jnp.float32)
    gi = jnp.dot(ih[...].T, ih[...], precision=lax.Precision.HIGHEST,
                 preferred_element_type=jnp.float32)
    g = jnp.where(i == 0, gu, gi)
    out_h[...] = jnp.dot(ssum, g, precision=lax.Precision.HIGHEST,
                         preferred_element_type=jnp.float32)


_final = pl.pallas_call(
    _final_body,
    grid=(2,),
    in_specs=[
        pl.BlockSpec((_USER, _D), lambda i: (i, 0)),
        pl.BlockSpec((_USER, _D), lambda i: (i, 0)),
        pl.BlockSpec((2, _USER, _D), lambda i: (0, i, 0)),
        pl.BlockSpec((_D, _D), lambda i: (0, 0)),
        pl.BlockSpec((_D, _D), lambda i: (0, 0)),
    ],
    out_specs=[
        pl.BlockSpec((_USER, _D), lambda i: (i, 0)),
        pl.BlockSpec((_USER, _D), lambda i: (i, 0)),
    ],
    out_shape=[
        jax.ShapeDtypeStruct((_N, _D), jnp.float32),
        jax.ShapeDtypeStruct((_N, _D), jnp.float32),
    ],
)


@jax.jit
def kernel(edge_index, edge_weight, uEmbeds, iEmbeds, uHyper, iHyper):
    table0 = jnp.concatenate([uEmbeds, iEmbeds], axis=0)
    rows = edge_index[0].astype(jnp.int32).reshape(_NW, _NSTG, _CPS, _CH)
    cols = edge_index[1].astype(jnp.int32).reshape(_NW, _NSTG, _CPS, _CH)
    wr = edge_weight.reshape(_NW, _NSTG, _CPS, _CH)

    p1 = _spmm(table0, cols, rows, wr)
    lat1 = _combine(p1)
    p2 = _spmm(lat1, cols, rows, wr)
    out_e, out_h = _final(table0, lat1, p2, uHyper, iHyper)
    return out_e, out_h[:_USER], out_h[_USER:]
